# baseline (speedup 1.0000x reference)
import jax
import jax.numpy as jnp
from jax.experimental import pallas as pl
from jax.experimental.pallas import tpu as pltpu

_EMB = 128
_NF = 9
_B = 2048
_NPAD = 102400


def _body(rows01_ref, k_ref, o_ref):
    base = jnp.sum(rows01_ref[:, 0, :], axis=0)
    d = rows01_ref[:, 1, :] - rows01_ref[:, 0, :]
    d16 = jnp.concatenate([d, jnp.zeros((7, _EMB), jnp.float32)], axis=0)
    d_hi = d16.astype(jnp.bfloat16)
    d_lo = (d16 - d_hi.astype(jnp.float32)).astype(jnp.bfloat16)
    kb = k_ref[...]
    ii = jax.lax.broadcasted_iota(jnp.int32, (16, _EMB), 0)
    pieces = []
    for r in range(16):
        row = jnp.broadcast_to(kb[r : r + 1, :], (16, _EMB))
        pieces.append((row >> ii) & 1)
    xt = jnp.concatenate(pieces, axis=1)
    xb = xt.astype(jnp.bfloat16)
    dn = (((0,), (0,)), ((), ()))
    acc = jax.lax.dot_general(xb, d_hi, dn, preferred_element_type=jnp.float32)
    acc = acc + jax.lax.dot_general(xb, d_lo, dn, preferred_element_type=jnp.float32)
    o_ref[...] = acc + base[None, :]


def kernel(x, W0, W1, W2, W3, W4, W5, W6, W7, W8):
    n = x.shape[0]
    rows01 = jnp.stack([W[:2] for W in (W0, W1, W2, W3, W4, W5, W6, W7, W8)])
    pow2 = (1 << jnp.arange(_NF, dtype=jnp.int32))[None, :]
    keys = jnp.sum(x * pow2, axis=1, dtype=jnp.int32)
    keys = jnp.pad(keys, (0, _NPAD - n)).reshape(_NPAD // 128, 128)
    grid = pl.cdiv(n, _B)
    return pl.pallas_call(
        _body,
        grid=(grid,),
        in_specs=[
            pl.BlockSpec((_NF, 2, _EMB), lambda i: (0, 0, 0)),
            pl.BlockSpec((16, 128), lambda i: (i, 0)),
        ],
        out_specs=pl.BlockSpec((_B, _EMB), lambda i: (i, 0)),
        out_shape=jax.ShapeDtypeStruct((n, _EMB), jnp.float32),
    )(rows01, keys)
